# baseline (speedup 1.0000x reference)
import functools

import jax
import jax.numpy as jnp
from jax import lax
from jax.experimental import pallas as pl
from jax.experimental.pallas import tpu as pltpu
from jax.experimental.pallas import tpu_sc as plsc

NC, NS = 2, 16
NW = NC * NS
E_PAD = 102400
NCHUNK = E_PAD // NS // 128
SENT = 1 << 20
MT = 512
R = 5120
RACC = R + 128
SHARE = RACC // NS


@functools.lru_cache(maxsize=None)
def _make_seg(passes):
  mesh = plsc.VectorSubcoreMesh(core_axis_name="c", subcore_axis_name="s",
                                num_cores=NC, num_subcores=NS)
  scratch = [
      pltpu.VMEM((NCHUNK, 128), jnp.int32),
      pltpu.VMEM((NCHUNK, 128), jnp.int32),
      pltpu.VMEM((NCHUNK, 128), jnp.int32),
      pltpu.VMEM((128, 128), jnp.float32),
      pltpu.VMEM((128, 128), jnp.float32),
      pltpu.VMEM_SHARED((RACC, 128), jnp.float32),
      pltpu.SemaphoreType.DMA,
      pltpu.SemaphoreType.DMA,
      pltpu.SemaphoreType.DMA,
      pltpu.SemaphoreType.DMA,
  ]

  def body(x_hbm, src_hbm, dst_hbm, z128_hbm, out_hbm,
           srcv, dstv, dloc, gbuf0, gbuf1, acc, sg0, sg1, ss0, ss1):
    cid = lax.axis_index("c")
    sid = lax.axis_index("s")
    pltpu.sync_copy(src_hbm.at[sid], srcv)
    pltpu.sync_copy(dst_hbm.at[sid], dstv)

    def do_pass(p, carry):
      base = (p * NC + cid) * R
      row0 = sid * SHARE
      off, rem = 0, SHARE
      while rem > 0:
        nr = min(128, rem)
        pltpu.sync_copy(z128_hbm.at[pl.ds(0, nr)],
                        acc.at[pl.ds(row0 + off, nr)])
        off += nr
        rem -= nr
      plsc.subcore_barrier()
      for j in range(NCHUNK):
        for v in range(8):
          sl = pl.ds(v * 16, 16)
          d = dstv[j, sl]
          ok = (d >= base) & (d < base + R)
          dloc[j, sl] = jnp.where(ok, d - base, R)
      gbufs = (gbuf0, gbuf1)
      sgs = (sg0, sg1)
      sss = (ss0, ss1)
      gd = [None, None]
      sd = [None, None]
      gd[0] = pltpu.async_copy(x_hbm.at[srcv.at[0]], gbuf0, sg0)
      for j in range(NCHUNK):
        b = j % 2
        gd[b].wait()
        sd[b] = pltpu.async_copy(gbufs[b], acc.at[dloc.at[j]], sss[b],
                                 add=True)
        if j + 1 < NCHUNK:
          if sd[1 - b] is not None:
            sd[1 - b].wait()
          gd[1 - b] = pltpu.async_copy(x_hbm.at[srcv.at[j + 1]],
                                       gbufs[1 - b], sgs[1 - b])
      sd[(NCHUNK - 1) % 2].wait()
      if sd[NCHUNK % 2] is not None:
        sd[NCHUNK % 2].wait()
      plsc.subcore_barrier()
      pltpu.sync_copy(acc.at[pl.ds(row0, SHARE)],
                      out_hbm.at[p, cid, pl.ds(row0, SHARE)])
      return carry

    lax.fori_loop(0, passes, do_pass, 0)

  return pl.kernel(
      body,
      out_type=(jax.ShapeDtypeStruct((passes, NC, RACC, 128), jnp.float32),),
      mesh=mesh, scratch_types=scratch)


@functools.lru_cache(maxsize=None)
def _make_dense(n_rows, nrel, d_out):
  bpp = R // MT
  nb = -(-n_rows // MT)
  readout = d_out is not None

  def sum_map(i):
    return (i // (2 * bpp), (i // bpp) % 2, i % bpp, 0)

  def body(*refs):
    idx = 0
    sums, cnts = [], []
    for _ in range(nrel):
      sums.append(refs[idx][0, 0])
      cnts.append(refs[idx + 1][0, 0])
      idx += 2
    x = refs[idx]; idx += 1
    wls = [refs[idx + k] for k in range(nrel)]; idx += nrel
    wr = refs[idx]; b = refs[idx + 1]; idx += 2
    if readout:
      wro = refs[idx]; bro = refs[idx + 1]; idx += 2
    o_ref = refs[idx]
    acc = jnp.dot(x[...], wr[...], preferred_element_type=jnp.float32)
    acc += b[...]
    for k in range(nrel):
      cnt = jnp.maximum(cnts[k][:, 0:1], 1.0)
      mean = sums[k] / cnt
      acc += jnp.dot(mean, wls[k][...], preferred_element_type=jnp.float32)
    h = jnp.maximum(acc, 0.0)
    if readout:
      h = jnp.dot(h, wro[...], preferred_element_type=jnp.float32) + bro[...]
    o_ref[...] = h

  in_specs = []
  for _ in range(nrel):
    in_specs.append(pl.BlockSpec((1, 1, MT, 128), sum_map))
    in_specs.append(pl.BlockSpec((1, 1, MT, 128), sum_map))
  in_specs.append(pl.BlockSpec((MT, 128), lambda i: (i, 0)))
  for _ in range(nrel):
    in_specs.append(pl.BlockSpec((128, 128), lambda i: (0, 0)))
  in_specs.append(pl.BlockSpec((128, 128), lambda i: (0, 0)))
  in_specs.append(pl.BlockSpec((1, 128), lambda i: (0, 0)))
  if readout:
    in_specs.append(pl.BlockSpec((128, d_out), lambda i: (0, 0)))
    in_specs.append(pl.BlockSpec((1, d_out), lambda i: (0, 0)))
  width = d_out if readout else 128
  return pl.pallas_call(
      body,
      grid=(nb,),
      in_specs=in_specs,
      out_specs=pl.BlockSpec((MT, width), lambda i: (i, 0)),
      out_shape=jax.ShapeDtypeStruct((n_rows, width), jnp.float32),
  )


def _prep_edges(ei):
  e = ei.shape[1]
  src = jnp.pad(ei[0], (0, E_PAD - e)).reshape(NS, NCHUNK, 128)
  dst = jnp.pad(ei[1], (0, E_PAD - e), constant_values=SENT)
  return src, dst.reshape(NS, NCHUNK, 128)


def kernel(x_coedge, x_face, x_edge, ei_next, ei_mate, ei_to_face, ei_to_edge,
           params):
  p = params
  z128 = jnp.zeros((128, 128), jnp.float32)
  ones_tab = jnp.ones((8, 128), jnp.float32)
  src0 = jnp.zeros((NS, NCHUNK, 128), jnp.int32)

  sn, dn = _prep_edges(ei_next)
  sm, dm = _prep_edges(ei_mate)
  sf, df = _prep_edges(ei_to_face)
  se, de = _prep_edges(ei_to_edge)

  seg10 = _make_seg(10)
  seg5 = _make_seg(5)
  seg1 = _make_seg(1)

  def chain(dep, arr):
    return arr + (dep[0, 0, 0, 0].astype(jnp.int32) * 0)

  (c_n,) = seg10(ones_tab, src0, dn, z128)
  (c_m,) = seg10(ones_tab, chain(c_n, src0), dm, z128)
  (c_f,) = seg1(ones_tab, chain(c_m, src0), df, z128)
  (c_e,) = seg5(ones_tab, chain(c_f, src0), de, z128)

  (s_n,) = seg10(x_coedge, chain(c_e, sn), dn, z128)
  (s_m,) = seg10(x_coedge, chain(s_n, sm), dm, z128)
  (s_f,) = seg1(x_coedge, chain(s_m, sf), df, z128)
  (s_e,) = seg5(x_coedge, chain(s_f, se), de, z128)

  dense_co1 = _make_dense(100000, 2, None)
  dense_fa1 = _make_dense(10000, 1, None)
  dense_ed1 = _make_dense(50000, 1, None)
  h_co = dense_co1(s_n, c_n, s_m, c_m, x_coedge,
                   p['W1_next_l'], p['W1_mate_l'],
                   p['W1_next_r'] + p['W1_mate_r'],
                   (p['b1_next'] + p['b1_mate']).reshape(1, 128))
  h_fa = dense_fa1(s_f, c_f, x_face, p['W1_to_face_l'], p['W1_to_face_r'],
                   p['b1_to_face'].reshape(1, 128))
  h_ed = dense_ed1(s_e, c_e, x_edge, p['W1_to_edge_l'], p['W1_to_edge_r'],
                   p['b1_to_edge'].reshape(1, 128))

  (t_n,) = seg10(h_co, chain(s_e, sn), dn, z128)
  (t_m,) = seg10(h_co, chain(t_n, sm), dm, z128)
  (t_f,) = seg1(h_co, chain(t_m, sf), df, z128)
  (t_e,) = seg5(h_co, chain(t_f, se), de, z128)

  dense_co2 = _make_dense(100000, 2, 256)
  dense_fa2 = _make_dense(10000, 1, 256)
  dense_ed2 = _make_dense(50000, 1, 256)
  z_co = dense_co2(t_n, c_n, t_m, c_m, h_co,
                   p['W2_next_l'], p['W2_mate_l'],
                   p['W2_next_r'] + p['W2_mate_r'],
                   (p['b2_next'] + p['b2_mate']).reshape(1, 128),
                   p['Wro_coedge'], p['bro_coedge'].reshape(1, 256))
  z_fa = dense_fa2(t_f, c_f, h_fa, p['W2_to_face_l'], p['W2_to_face_r'],
                   p['b2_to_face'].reshape(1, 128),
                   p['Wro_face'], p['bro_face'].reshape(1, 256))
  z_ed = dense_ed2(t_e, c_e, h_ed, p['W2_to_edge_l'], p['W2_to_edge_r'],
                   p['b2_to_edge'].reshape(1, 128),
                   p['Wro_edge'], p['bro_edge'].reshape(1, 256))
  return (z_co, z_fa, z_ed)


# speedup vs baseline: 1.0003x; 1.0003x over previous
import functools

import jax
import jax.numpy as jnp
from jax import lax
from jax.experimental import pallas as pl
from jax.experimental.pallas import tpu as pltpu
from jax.experimental.pallas import tpu_sc as plsc

NC, NS = 2, 16
NW = NC * NS
E_PAD = 102400
NCHUNK = E_PAD // NS // 128
SENT = 1 << 20
MT = 512
R = 5120
RACC = R + 128
SHARE = RACC // NS


@functools.lru_cache(maxsize=None)
def _make_seg(passes):
  mesh = plsc.VectorSubcoreMesh(core_axis_name="c", subcore_axis_name="s",
                                num_cores=NC, num_subcores=NS)
  scratch = [
      pltpu.VMEM((NCHUNK, 128), jnp.int32),
      pltpu.VMEM((NCHUNK, 128), jnp.int32),
      pltpu.VMEM((NCHUNK, 128), jnp.int32),
      pltpu.VMEM((128, 128), jnp.float32),
      pltpu.VMEM((128, 128), jnp.float32),
      pltpu.VMEM_SHARED((RACC, 128), jnp.float32),
      pltpu.SemaphoreType.DMA,
      pltpu.SemaphoreType.DMA,
      pltpu.SemaphoreType.DMA,
      pltpu.SemaphoreType.DMA,
  ]

  def body(x_hbm, src_hbm, dst_hbm, z128_hbm, out_hbm,
           srcv, dstv, dloc, gbuf0, gbuf1, acc, sg0, sg1, ss0, ss1):
    cid = lax.axis_index("c")
    sid = lax.axis_index("s")
    pltpu.sync_copy(src_hbm.at[sid], srcv)
    pltpu.sync_copy(dst_hbm.at[sid], dstv)

    def do_pass(p, carry):
      base = (p * NC + cid) * R
      row0 = sid * SHARE
      off, rem = 0, SHARE
      while rem > 0:
        nr = min(128, rem)
        pltpu.sync_copy(z128_hbm.at[pl.ds(0, nr)],
                        acc.at[pl.ds(row0 + off, nr)])
        off += nr
        rem -= nr
      plsc.subcore_barrier()
      for j in range(NCHUNK):
        for v in range(8):
          sl = pl.ds(v * 16, 16)
          d = dstv[j, sl]
          ok = (d >= base) & (d < base + R)
          dloc[j, sl] = jnp.where(ok, d - base, R)
      gbufs = (gbuf0, gbuf1)
      sgs = (sg0, sg1)
      sss = (ss0, ss1)
      gd = [None, None]
      sd = [None, None]
      gd[0] = pltpu.async_copy(x_hbm.at[srcv.at[0]], gbuf0, sg0)
      for j in range(NCHUNK):
        b = j % 2
        gd[b].wait()
        sd[b] = pltpu.async_copy(gbufs[b], acc.at[dloc.at[j]], sss[b],
                                 add=True)
        if j + 1 < NCHUNK:
          if sd[1 - b] is not None:
            sd[1 - b].wait()
          gd[1 - b] = pltpu.async_copy(x_hbm.at[srcv.at[j + 1]],
                                       gbufs[1 - b], sgs[1 - b])
      sd[(NCHUNK - 1) % 2].wait()
      if sd[NCHUNK % 2] is not None:
        sd[NCHUNK % 2].wait()
      plsc.subcore_barrier()
      pltpu.sync_copy(acc.at[pl.ds(row0, SHARE)],
                      out_hbm.at[p, cid, pl.ds(row0, SHARE)])
      return carry

    lax.fori_loop(0, passes, do_pass, 0)

  return pl.kernel(
      body,
      out_type=(jax.ShapeDtypeStruct((passes, NC, RACC, 128), jnp.float32),),
      mesh=mesh, scratch_types=scratch)


@functools.lru_cache(maxsize=None)
def _make_dense(n_rows, nrel, d_out):
  bpp = R // MT
  nb = -(-n_rows // MT)
  readout = d_out is not None

  def sum_map(i):
    return (i // (2 * bpp), (i // bpp) % 2, i % bpp, 0)

  def body(*refs):
    idx = 0
    sums, cnts = [], []
    for _ in range(nrel):
      sums.append(refs[idx][0, 0])
      cnts.append(refs[idx + 1][0, 0])
      idx += 2
    x = refs[idx]; idx += 1
    wls = [refs[idx + k] for k in range(nrel)]; idx += nrel
    wr = refs[idx]; b = refs[idx + 1]; idx += 2
    if readout:
      wro = refs[idx]; bro = refs[idx + 1]; idx += 2
    o_ref = refs[idx]
    acc = jnp.dot(x[...], wr[...], preferred_element_type=jnp.float32)
    acc += b[...]
    for k in range(nrel):
      cnt = jnp.maximum(cnts[k][:, 0:1], 1.0)
      mean = sums[k] / cnt
      acc += jnp.dot(mean, wls[k][...], preferred_element_type=jnp.float32)
    h = jnp.maximum(acc, 0.0)
    if readout:
      h = jnp.dot(h, wro[...], preferred_element_type=jnp.float32) + bro[...]
    o_ref[...] = h

  in_specs = []
  for _ in range(nrel):
    in_specs.append(pl.BlockSpec((1, 1, MT, 128), sum_map))
    in_specs.append(pl.BlockSpec((1, 1, MT, 128), sum_map))
  in_specs.append(pl.BlockSpec((MT, 128), lambda i: (i, 0)))
  for _ in range(nrel):
    in_specs.append(pl.BlockSpec((128, 128), lambda i: (0, 0)))
  in_specs.append(pl.BlockSpec((128, 128), lambda i: (0, 0)))
  in_specs.append(pl.BlockSpec((1, 128), lambda i: (0, 0)))
  if readout:
    in_specs.append(pl.BlockSpec((128, d_out), lambda i: (0, 0)))
    in_specs.append(pl.BlockSpec((1, d_out), lambda i: (0, 0)))
  width = d_out if readout else 128
  return pl.pallas_call(
      body,
      grid=(nb,),
      in_specs=in_specs,
      out_specs=pl.BlockSpec((MT, width), lambda i: (i, 0)),
      out_shape=jax.ShapeDtypeStruct((n_rows, width), jnp.float32),
  )


def _prep_edges(ei):
  e = ei.shape[1]
  src = jnp.pad(ei[0], (0, E_PAD - e)).reshape(NS, NCHUNK, 128)
  dst = jnp.pad(ei[1], (0, E_PAD - e), constant_values=SENT)
  return src, dst.reshape(NS, NCHUNK, 128)


def kernel(x_coedge, x_face, x_edge, ei_next, ei_mate, ei_to_face, ei_to_edge,
           params):
  p = params
  z128 = jnp.zeros((128, 128), jnp.float32)
  ones_tab = jnp.ones((8, 128), jnp.float32)
  src0 = jnp.zeros((NS, NCHUNK, 128), jnp.int32)

  sn, dn = _prep_edges(ei_next)
  sm, dm = _prep_edges(ei_mate)
  sf, df = _prep_edges(ei_to_face)
  se, de = _prep_edges(ei_to_edge)

  seg10 = _make_seg(10)
  seg5 = _make_seg(5)
  seg1 = _make_seg(1)

  def chain(dep, arr):
    arr2, _ = lax.optimization_barrier((arr, dep[0, 0, 0, 0]))
    return arr2

  (c_n,) = seg10(ones_tab, src0, dn, z128)
  (c_m,) = seg10(ones_tab, chain(c_n, src0), dm, z128)
  (c_f,) = seg1(ones_tab, chain(c_m, src0), df, z128)
  (c_e,) = seg5(ones_tab, chain(c_f, src0), de, z128)

  (s_n,) = seg10(x_coedge, chain(c_e, sn), dn, z128)
  (s_m,) = seg10(x_coedge, chain(s_n, sm), dm, z128)
  (s_f,) = seg1(x_coedge, chain(s_m, sf), df, z128)
  (s_e,) = seg5(x_coedge, chain(s_f, se), de, z128)

  dense_co1 = _make_dense(100000, 2, None)
  dense_fa1 = _make_dense(10000, 1, None)
  dense_ed1 = _make_dense(50000, 1, None)
  h_co = dense_co1(s_n, c_n, s_m, c_m, x_coedge,
                   p['W1_next_l'], p['W1_mate_l'],
                   p['W1_next_r'] + p['W1_mate_r'],
                   (p['b1_next'] + p['b1_mate']).reshape(1, 128))
  h_fa = dense_fa1(s_f, c_f, x_face, p['W1_to_face_l'], p['W1_to_face_r'],
                   p['b1_to_face'].reshape(1, 128))
  h_ed = dense_ed1(s_e, c_e, x_edge, p['W1_to_edge_l'], p['W1_to_edge_r'],
                   p['b1_to_edge'].reshape(1, 128))

  (t_n,) = seg10(h_co, chain(s_e, sn), dn, z128)
  (t_m,) = seg10(h_co, chain(t_n, sm), dm, z128)
  (t_f,) = seg1(h_co, chain(t_m, sf), df, z128)
  (t_e,) = seg5(h_co, chain(t_f, se), de, z128)

  dense_co2 = _make_dense(100000, 2, 256)
  dense_fa2 = _make_dense(10000, 1, 256)
  dense_ed2 = _make_dense(50000, 1, 256)
  z_co = dense_co2(t_n, c_n, t_m, c_m, h_co,
                   p['W2_next_l'], p['W2_mate_l'],
                   p['W2_next_r'] + p['W2_mate_r'],
                   (p['b2_next'] + p['b2_mate']).reshape(1, 128),
                   p['Wro_coedge'], p['bro_coedge'].reshape(1, 256))
  z_fa = dense_fa2(t_f, c_f, h_fa, p['W2_to_face_l'], p['W2_to_face_r'],
                   p['b2_to_face'].reshape(1, 128),
                   p['Wro_face'], p['bro_face'].reshape(1, 256))
  z_ed = dense_ed2(t_e, c_e, h_ed, p['W2_to_edge_l'], p['W2_to_edge_r'],
                   p['b2_to_edge'].reshape(1, 128),
                   p['Wro_edge'], p['bro_edge'].reshape(1, 256))
  return (z_co, z_fa, z_ed)


# speedup vs baseline: 12.3560x; 12.3523x over previous
import functools

import jax
import jax.numpy as jnp
from jax import lax
from jax.experimental import pallas as pl
from jax.experimental.pallas import tpu as pltpu
from jax.experimental.pallas import tpu_sc as plsc

NC, NS = 2, 16
NW = NC * NS
E_PAD = 102400
NCHUNK = E_PAD // NS // 128
SENT = 1 << 20
MT = 512
R = 5120
RACC = R + 128
SHARE = RACC // NS


@functools.lru_cache(maxsize=None)
def _make_seg(passes):
  mesh = plsc.VectorSubcoreMesh(core_axis_name="c", subcore_axis_name="s",
                                num_cores=NC, num_subcores=NS)
  scratch = [
      pltpu.VMEM((NCHUNK, 128), jnp.int32),
      pltpu.VMEM((NCHUNK, 128), jnp.int32),
      pltpu.VMEM((NCHUNK, 128), jnp.int32),
      pltpu.VMEM((128, 128), jnp.float32),
      pltpu.VMEM((128, 128), jnp.float32),
      pltpu.VMEM_SHARED((RACC, 128), jnp.float32),
      pltpu.SemaphoreType.DMA,
      pltpu.SemaphoreType.DMA,
      pltpu.SemaphoreType.DMA,
      pltpu.SemaphoreType.DMA,
  ]

  def body(x_hbm, src_hbm, dst_hbm, z128_hbm, out_hbm,
           srcv, dstv, dloc, gbuf0, gbuf1, acc, sg0, sg1, ss0, ss1):
    cid = lax.axis_index("c")
    sid = lax.axis_index("s")
    pltpu.sync_copy(src_hbm.at[sid], srcv)
    pltpu.sync_copy(dst_hbm.at[sid], dstv)

    def do_pass(p, carry):
      base = (p * NC + cid) * R
      row0 = sid * SHARE
      off, rem = 0, SHARE
      while rem > 0:
        nr = min(128, rem)
        pltpu.sync_copy(z128_hbm.at[pl.ds(0, nr)],
                        acc.at[pl.ds(row0 + off, nr)])
        off += nr
        rem -= nr
      plsc.subcore_barrier()
      for j in range(NCHUNK):
        for v in range(8):
          sl = pl.ds(v * 16, 16)
          d = dstv[j, sl]
          ok = (d >= base) & (d < base + R)
          dloc[j, sl] = jnp.where(ok, d - base, R)
      gbufs = (gbuf0, gbuf1)
      sgs = (sg0, sg1)
      sss = (ss0, ss1)
      gd = [None, None]
      sd = [None, None]
      gd[0] = pltpu.async_copy(x_hbm.at[srcv.at[0]], gbuf0, sg0)
      for j in range(NCHUNK):
        b = j % 2
        gd[b].wait()
        sd[b] = pltpu.async_copy(gbufs[b], acc.at[dloc.at[j]], sss[b],
                                 add=True)
        if j + 1 < NCHUNK:
          if sd[1 - b] is not None:
            sd[1 - b].wait()
          gd[1 - b] = pltpu.async_copy(x_hbm.at[srcv.at[j + 1]],
                                       gbufs[1 - b], sgs[1 - b])
      sd[(NCHUNK - 1) % 2].wait()
      if sd[NCHUNK % 2] is not None:
        sd[NCHUNK % 2].wait()
      plsc.subcore_barrier()
      pltpu.sync_copy(acc.at[pl.ds(row0, SHARE)],
                      out_hbm.at[p, cid, pl.ds(row0, SHARE)])
      return carry

    lax.fori_loop(0, passes, do_pass, 0)

  return pl.kernel(
      body,
      out_type=(jax.ShapeDtypeStruct((passes, NC, RACC, 128), jnp.float32),),
      mesh=mesh, scratch_types=scratch)


@functools.lru_cache(maxsize=None)
def _make_cnt(passes):
  mesh = plsc.VectorSubcoreMesh(core_axis_name="c", subcore_axis_name="s",
                                num_cores=NC, num_subcores=NS)
  scratch = [
      pltpu.VMEM((NCHUNK, 128), jnp.int32),
      pltpu.VMEM((NCHUNK, 128), jnp.int32),
      pltpu.VMEM((128, 128), jnp.float32),
      pltpu.VMEM_SHARED((RACC, 128), jnp.float32),
  ]

  def body(dst_hbm, one_hbm, z128_hbm, out_hbm, dstv, dloc, obuf, acc):
    cid = lax.axis_index("c")
    sid = lax.axis_index("s")
    pltpu.sync_copy(dst_hbm.at[sid], dstv)
    pltpu.sync_copy(one_hbm, obuf)

    def do_pass(p, carry):
      base = (p * NC + cid) * R
      row0 = sid * SHARE
      off, rem = 0, SHARE
      while rem > 0:
        nr = min(128, rem)
        pltpu.sync_copy(z128_hbm.at[pl.ds(0, nr)],
                        acc.at[pl.ds(row0 + off, nr)])
        off += nr
        rem -= nr
      plsc.subcore_barrier()
      for j in range(NCHUNK):
        for v in range(8):
          sl = pl.ds(v * 16, 16)
          d = dstv[j, sl]
          ok = (d >= base) & (d < base + R)
          dloc[j, sl] = jnp.where(ok, d - base, R)
      for j in range(NCHUNK):
        pltpu.sync_copy(obuf, acc.at[dloc.at[j]], add=True)
      plsc.subcore_barrier()
      pltpu.sync_copy(acc.at[pl.ds(row0, SHARE)],
                      out_hbm.at[p, cid, pl.ds(row0, SHARE)])
      return carry

    lax.fori_loop(0, passes, do_pass, 0)

  return pl.kernel(
      body,
      out_type=(jax.ShapeDtypeStruct((passes, NC, RACC, 128), jnp.float32),),
      mesh=mesh, scratch_types=scratch)


@functools.lru_cache(maxsize=None)
def _make_dense(n_rows, nrel, d_out):
  bpp = R // MT
  nb = -(-n_rows // MT)
  readout = d_out is not None

  def sum_map(i):
    return (i // (2 * bpp), (i // bpp) % 2, i % bpp, 0)

  def body(*refs):
    idx = 0
    sums, cnts = [], []
    for _ in range(nrel):
      sums.append(refs[idx][0, 0])
      cnts.append(refs[idx + 1][0, 0])
      idx += 2
    x = refs[idx]; idx += 1
    wls = [refs[idx + k] for k in range(nrel)]; idx += nrel
    wr = refs[idx]; b = refs[idx + 1]; idx += 2
    if readout:
      wro = refs[idx]; bro = refs[idx + 1]; idx += 2
    o_ref = refs[idx]
    acc = jnp.dot(x[...], wr[...], preferred_element_type=jnp.float32)
    acc += b[...]
    for k in range(nrel):
      cnt = jnp.maximum(cnts[k][:, 0:1], 1.0)
      mean = sums[k] / cnt
      acc += jnp.dot(mean, wls[k][...], preferred_element_type=jnp.float32)
    h = jnp.maximum(acc, 0.0)
    if readout:
      h = jnp.dot(h, wro[...], preferred_element_type=jnp.float32) + bro[...]
    o_ref[...] = h

  in_specs = []
  for _ in range(nrel):
    in_specs.append(pl.BlockSpec((1, 1, MT, 128), sum_map))
    in_specs.append(pl.BlockSpec((1, 1, MT, 128), sum_map))
  in_specs.append(pl.BlockSpec((MT, 128), lambda i: (i, 0)))
  for _ in range(nrel):
    in_specs.append(pl.BlockSpec((128, 128), lambda i: (0, 0)))
  in_specs.append(pl.BlockSpec((128, 128), lambda i: (0, 0)))
  in_specs.append(pl.BlockSpec((1, 128), lambda i: (0, 0)))
  if readout:
    in_specs.append(pl.BlockSpec((128, d_out), lambda i: (0, 0)))
    in_specs.append(pl.BlockSpec((1, d_out), lambda i: (0, 0)))
  width = d_out if readout else 128
  return pl.pallas_call(
      body,
      grid=(nb,),
      in_specs=in_specs,
      out_specs=pl.BlockSpec((MT, width), lambda i: (i, 0)),
      out_shape=jax.ShapeDtypeStruct((n_rows, width), jnp.float32),
  )


def _prep_edges(ei):
  e = ei.shape[1]
  src = jnp.pad(ei[0], (0, E_PAD - e)).reshape(NS, NCHUNK, 128)
  dst = jnp.pad(ei[1], (0, E_PAD - e), constant_values=SENT)
  return src, dst.reshape(NS, NCHUNK, 128)


def kernel(x_coedge, x_face, x_edge, ei_next, ei_mate, ei_to_face, ei_to_edge,
           params):
  p = params
  z128 = jnp.zeros((128, 128), jnp.float32)
  one128 = jnp.ones((128, 128), jnp.float32)

  sn, dn = _prep_edges(ei_next)
  sm, dm = _prep_edges(ei_mate)
  sf, df = _prep_edges(ei_to_face)
  se, de = _prep_edges(ei_to_edge)

  seg10 = _make_seg(10)
  seg5 = _make_seg(5)
  seg1 = _make_seg(1)

  cnt10 = _make_cnt(10)
  cnt5 = _make_cnt(5)
  cnt1 = _make_cnt(1)

  (c_n,) = cnt10(dn, one128, z128)
  (c_m,) = cnt10(dm, one128, z128)
  (c_f,) = cnt1(df, one128, z128)
  (c_e,) = cnt5(de, one128, z128)

  (s_n,) = seg10(x_coedge, sn, dn, z128)
  (s_m,) = seg10(x_coedge, sm, dm, z128)
  (s_f,) = seg1(x_coedge, sf, df, z128)
  (s_e,) = seg5(x_coedge, se, de, z128)

  dense_co1 = _make_dense(100000, 2, None)
  dense_fa1 = _make_dense(10000, 1, None)
  dense_ed1 = _make_dense(50000, 1, None)
  h_co = dense_co1(s_n, c_n, s_m, c_m, x_coedge,
                   p['W1_next_l'], p['W1_mate_l'],
                   p['W1_next_r'] + p['W1_mate_r'],
                   (p['b1_next'] + p['b1_mate']).reshape(1, 128))
  h_fa = dense_fa1(s_f, c_f, x_face, p['W1_to_face_l'], p['W1_to_face_r'],
                   p['b1_to_face'].reshape(1, 128))
  h_ed = dense_ed1(s_e, c_e, x_edge, p['W1_to_edge_l'], p['W1_to_edge_r'],
                   p['b1_to_edge'].reshape(1, 128))

  (t_n,) = seg10(h_co, sn, dn, z128)
  (t_m,) = seg10(h_co, sm, dm, z128)
  (t_f,) = seg1(h_co, sf, df, z128)
  (t_e,) = seg5(h_co, se, de, z128)

  dense_co2 = _make_dense(100000, 2, 256)
  dense_fa2 = _make_dense(10000, 1, 256)
  dense_ed2 = _make_dense(50000, 1, 256)
  z_co = dense_co2(t_n, c_n, t_m, c_m, h_co,
                   p['W2_next_l'], p['W2_mate_l'],
                   p['W2_next_r'] + p['W2_mate_r'],
                   (p['b2_next'] + p['b2_mate']).reshape(1, 128),
                   p['Wro_coedge'], p['bro_coedge'].reshape(1, 256))
  z_fa = dense_fa2(t_f, c_f, h_fa, p['W2_to_face_l'], p['W2_to_face_r'],
                   p['b2_to_face'].reshape(1, 128),
                   p['Wro_face'], p['bro_face'].reshape(1, 256))
  z_ed = dense_ed2(t_e, c_e, h_ed, p['W2_to_edge_l'], p['W2_to_edge_r'],
                   p['b2_to_edge'].reshape(1, 128),
                   p['Wro_edge'], p['bro_edge'].reshape(1, 256))
  return (z_co, z_fa, z_ed)


# speedup vs baseline: 16.3316x; 1.3218x over previous
import functools

import jax
import jax.numpy as jnp
from jax import lax
from jax.experimental import pallas as pl
from jax.experimental.pallas import tpu as pltpu
from jax.experimental.pallas import tpu_sc as plsc

NC, NS = 2, 16
NW = NC * NS
E_PAD = 102400
NCHUNK = E_PAD // NS // 128
SENT = 1 << 20
MT = 512
R = 7168
RACC = R + 128
SHARE = RACC // NS


@functools.lru_cache(maxsize=None)
def _make_seg(passes):
  mesh = plsc.VectorSubcoreMesh(core_axis_name="c", subcore_axis_name="s",
                                num_cores=NC, num_subcores=NS)
  scratch = [
      pltpu.VMEM((NCHUNK, 128), jnp.int32),
      pltpu.VMEM((NCHUNK, 128), jnp.int32),
      pltpu.VMEM((NCHUNK, 128), jnp.int32),
      pltpu.VMEM((128, 128), jnp.float32),
      pltpu.VMEM_SHARED((RACC, 128), jnp.float32),
      pltpu.SemaphoreType.DMA,
  ]

  def body(x_hbm, src_hbm, dst_hbm, z128_hbm, out_hbm,
           srcv, dstv, dloc, gbuf, acc, sem):
    cid = lax.axis_index("c")
    sid = lax.axis_index("s")
    pltpu.sync_copy(src_hbm.at[sid], srcv)
    pltpu.sync_copy(dst_hbm.at[sid], dstv)

    def do_pass(p, carry):
      base = (p * NC + cid) * R
      row0 = sid * SHARE
      off, rem = 0, SHARE
      while rem > 0:
        nr = min(128, rem)
        pltpu.sync_copy(z128_hbm.at[pl.ds(0, nr)],
                        acc.at[pl.ds(row0 + off, nr)])
        off += nr
        rem -= nr
      plsc.subcore_barrier()
      for j in range(NCHUNK):
        for v in range(8):
          sl = pl.ds(v * 16, 16)
          d = dstv[j, sl]
          ok = (d >= base) & (d < base + R)
          dloc[j, sl] = jnp.where(ok, d - base, R)
      for j in range(NCHUNK):
        pltpu.async_copy(x_hbm.at[srcv.at[j]], gbuf, sem).wait()
        pltpu.sync_copy(gbuf, acc.at[dloc.at[j]], add=True)
      plsc.subcore_barrier()
      pltpu.sync_copy(acc.at[pl.ds(row0, SHARE)],
                      out_hbm.at[p, cid, pl.ds(row0, SHARE)])
      return carry

    lax.fori_loop(0, passes, do_pass, 0)

  return pl.kernel(
      body,
      out_type=(jax.ShapeDtypeStruct((passes, NC, RACC, 128), jnp.float32),),
      mesh=mesh, scratch_types=scratch)


@functools.lru_cache(maxsize=None)
def _make_cnt(passes):
  mesh = plsc.VectorSubcoreMesh(core_axis_name="c", subcore_axis_name="s",
                                num_cores=NC, num_subcores=NS)
  scratch = [
      pltpu.VMEM((NCHUNK, 128), jnp.int32),
      pltpu.VMEM((NCHUNK, 128), jnp.int32),
      pltpu.VMEM((128, 128), jnp.float32),
      pltpu.VMEM_SHARED((RACC, 128), jnp.float32),
  ]

  def body(dst_hbm, one_hbm, z128_hbm, out_hbm, dstv, dloc, obuf, acc):
    cid = lax.axis_index("c")
    sid = lax.axis_index("s")
    pltpu.sync_copy(dst_hbm.at[sid], dstv)
    pltpu.sync_copy(one_hbm, obuf)

    def do_pass(p, carry):
      base = (p * NC + cid) * R
      row0 = sid * SHARE
      off, rem = 0, SHARE
      while rem > 0:
        nr = min(128, rem)
        pltpu.sync_copy(z128_hbm.at[pl.ds(0, nr)],
                        acc.at[pl.ds(row0 + off, nr)])
        off += nr
        rem -= nr
      plsc.subcore_barrier()
      for j in range(NCHUNK):
        for v in range(8):
          sl = pl.ds(v * 16, 16)
          d = dstv[j, sl]
          ok = (d >= base) & (d < base + R)
          dloc[j, sl] = jnp.where(ok, d - base, R)
      for j in range(NCHUNK):
        pltpu.sync_copy(obuf, acc.at[dloc.at[j]], add=True)
      plsc.subcore_barrier()
      pltpu.sync_copy(acc.at[pl.ds(row0, SHARE)],
                      out_hbm.at[p, cid, pl.ds(row0, SHARE)])
      return carry

    lax.fori_loop(0, passes, do_pass, 0)

  return pl.kernel(
      body,
      out_type=(jax.ShapeDtypeStruct((passes, NC, RACC, 128), jnp.float32),),
      mesh=mesh, scratch_types=scratch)


@functools.lru_cache(maxsize=None)
def _make_dense(n_rows, nrel, d_out):
  bpp = R // MT
  nb = -(-n_rows // MT)
  readout = d_out is not None

  def sum_map(i):
    return (i // (2 * bpp), (i // bpp) % 2, i % bpp, 0)

  def body(*refs):
    idx = 0
    sums, cnts = [], []
    for _ in range(nrel):
      sums.append(refs[idx][0, 0])
      cnts.append(refs[idx + 1][0, 0])
      idx += 2
    x = refs[idx]; idx += 1
    wls = [refs[idx + k] for k in range(nrel)]; idx += nrel
    wr = refs[idx]; b = refs[idx + 1]; idx += 2
    if readout:
      wro = refs[idx]; bro = refs[idx + 1]; idx += 2
    o_ref = refs[idx]
    acc = jnp.dot(x[...], wr[...], preferred_element_type=jnp.float32)
    acc += b[...]
    for k in range(nrel):
      cnt = jnp.maximum(cnts[k][:, 0:1], 1.0)
      mean = sums[k] / cnt
      acc += jnp.dot(mean, wls[k][...], preferred_element_type=jnp.float32)
    h = jnp.maximum(acc, 0.0)
    if readout:
      h = jnp.dot(h, wro[...], preferred_element_type=jnp.float32) + bro[...]
    o_ref[...] = h

  in_specs = []
  for _ in range(nrel):
    in_specs.append(pl.BlockSpec((1, 1, MT, 128), sum_map))
    in_specs.append(pl.BlockSpec((1, 1, MT, 128), sum_map))
  in_specs.append(pl.BlockSpec((MT, 128), lambda i: (i, 0)))
  for _ in range(nrel):
    in_specs.append(pl.BlockSpec((128, 128), lambda i: (0, 0)))
  in_specs.append(pl.BlockSpec((128, 128), lambda i: (0, 0)))
  in_specs.append(pl.BlockSpec((1, 128), lambda i: (0, 0)))
  if readout:
    in_specs.append(pl.BlockSpec((128, d_out), lambda i: (0, 0)))
    in_specs.append(pl.BlockSpec((1, d_out), lambda i: (0, 0)))
  width = d_out if readout else 128
  return pl.pallas_call(
      body,
      grid=(nb,),
      in_specs=in_specs,
      out_specs=pl.BlockSpec((MT, width), lambda i: (i, 0)),
      out_shape=jax.ShapeDtypeStruct((n_rows, width), jnp.float32),
  )


def _prep_edges(ei):
  e = ei.shape[1]
  src = jnp.pad(ei[0], (0, E_PAD - e)).reshape(NS, NCHUNK, 128)
  dst = jnp.pad(ei[1], (0, E_PAD - e), constant_values=SENT)
  return src, dst.reshape(NS, NCHUNK, 128)


def kernel(x_coedge, x_face, x_edge, ei_next, ei_mate, ei_to_face, ei_to_edge,
           params):
  p = params
  z128 = jnp.zeros((128, 128), jnp.float32)
  one128 = jnp.ones((128, 128), jnp.float32)

  sn, dn = _prep_edges(ei_next)
  sm, dm = _prep_edges(ei_mate)
  sf, df = _prep_edges(ei_to_face)
  se, de = _prep_edges(ei_to_edge)

  seg10 = _make_seg(7)
  seg5 = _make_seg(4)
  seg1 = _make_seg(1)

  cnt10 = _make_cnt(7)
  cnt5 = _make_cnt(4)
  cnt1 = _make_cnt(1)

  (c_n,) = cnt10(dn, one128, z128)
  (c_m,) = cnt10(dm, one128, z128)
  (c_f,) = cnt1(df, one128, z128)
  (c_e,) = cnt5(de, one128, z128)

  (s_n,) = seg10(x_coedge, sn, dn, z128)
  (s_m,) = seg10(x_coedge, sm, dm, z128)
  (s_f,) = seg1(x_coedge, sf, df, z128)
  (s_e,) = seg5(x_coedge, se, de, z128)

  dense_co1 = _make_dense(100000, 2, None)
  dense_fa1 = _make_dense(10000, 1, None)
  dense_ed1 = _make_dense(50000, 1, None)
  h_co = dense_co1(s_n, c_n, s_m, c_m, x_coedge,
                   p['W1_next_l'], p['W1_mate_l'],
                   p['W1_next_r'] + p['W1_mate_r'],
                   (p['b1_next'] + p['b1_mate']).reshape(1, 128))
  h_fa = dense_fa1(s_f, c_f, x_face, p['W1_to_face_l'], p['W1_to_face_r'],
                   p['b1_to_face'].reshape(1, 128))
  h_ed = dense_ed1(s_e, c_e, x_edge, p['W1_to_edge_l'], p['W1_to_edge_r'],
                   p['b1_to_edge'].reshape(1, 128))

  (t_n,) = seg10(h_co, sn, dn, z128)
  (t_m,) = seg10(h_co, sm, dm, z128)
  (t_f,) = seg1(h_co, sf, df, z128)
  (t_e,) = seg5(h_co, se, de, z128)

  dense_co2 = _make_dense(100000, 2, 256)
  dense_fa2 = _make_dense(10000, 1, 256)
  dense_ed2 = _make_dense(50000, 1, 256)
  z_co = dense_co2(t_n, c_n, t_m, c_m, h_co,
                   p['W2_next_l'], p['W2_mate_l'],
                   p['W2_next_r'] + p['W2_mate_r'],
                   (p['b2_next'] + p['b2_mate']).reshape(1, 128),
                   p['Wro_coedge'], p['bro_coedge'].reshape(1, 256))
  z_fa = dense_fa2(t_f, c_f, h_fa, p['W2_to_face_l'], p['W2_to_face_r'],
                   p['b2_to_face'].reshape(1, 128),
                   p['Wro_face'], p['bro_face'].reshape(1, 256))
  z_ed = dense_ed2(t_e, c_e, h_ed, p['W2_to_edge_l'], p['W2_to_edge_r'],
                   p['b2_to_edge'].reshape(1, 128),
                   p['Wro_edge'], p['bro_edge'].reshape(1, 256))
  return (z_co, z_fa, z_ed)
